# Initial kernel scaffold; baseline (speedup 1.0000x reference)
import jax
import jax.numpy as jnp
from jax.experimental import pallas as pl


def kernel(midis_out):
    raise NotImplementedError("write your pallas kernel here")


# speedup vs baseline: 4.0092x; 4.0092x over previous
import functools

import jax
import jax.numpy as jnp
from jax.experimental import pallas as pl

_LST = 64
_NB = 16


def _bisect_tau(z, axis):
    hi = jnp.max(z, axis=axis, keepdims=True)
    lo = hi - 1.0
    for _ in range(_NB):
        mid = 0.5 * (lo + hi)
        g = jnp.sum(jnp.maximum(z - mid, 0.0), axis=axis, keepdims=True)
        ge = g >= 1.0
        lo = jnp.where(ge, mid, lo)
        hi = jnp.where(ge, hi, mid)
    sup = (z > lo).astype(jnp.float32)
    c = jnp.sum(sup, axis=axis, keepdims=True)
    s = jnp.sum(z * sup, axis=axis, keepdims=True)
    return jnp.clip((s - 1.0) / c, lo, hi)


def _body(x_ref, o_ref, *, t_block):
    z = x_ref[0]
    n_insts = z.shape[0]
    tau_i = _bisect_tau(z, axis=0)
    pi = jnp.maximum(z - tau_i, 0.0)
    r = z.reshape(n_insts, t_block // _LST, _LST)
    tau_t = _bisect_tau(r, axis=2)
    pt = jnp.maximum(r - tau_t, 0.0).reshape(n_insts, t_block)
    o_ref[0] = pi * pt


def kernel(midis_out):
    batch, n_insts, time = midis_out.shape
    t_block = 2048
    if time % t_block:
        t_block = _LST
    grid = (batch, time // t_block)
    spec = pl.BlockSpec((1, n_insts, t_block), lambda b, t: (b, 0, t))
    return pl.pallas_call(
        functools.partial(_body, t_block=t_block),
        grid=grid,
        in_specs=[spec],
        out_specs=spec,
        out_shape=jax.ShapeDtypeStruct(midis_out.shape, midis_out.dtype),
    )(midis_out)


# speedup vs baseline: 10.7385x; 2.6785x over previous
import functools

import jax
import jax.numpy as jnp
from jax.experimental import pallas as pl

_LST = 64
_NB = 16


def _bisect_tau(z, axis):
    hi = jnp.max(z, axis=axis, keepdims=True)
    lo = hi - 1.0
    for _ in range(_NB):
        mid = 0.5 * (lo + hi)
        g = jnp.sum(jnp.maximum(z - mid, 0.0), axis=axis, keepdims=True)
        ge = g >= 1.0
        lo = jnp.where(ge, mid, lo)
        hi = jnp.where(ge, hi, mid)
    sup = (z > lo).astype(jnp.float32)
    c = jnp.sum(sup, axis=axis, keepdims=True)
    s = jnp.sum(z * sup, axis=axis, keepdims=True)
    return jnp.clip((s - 1.0) / c, lo, hi)


def _body(x_ref, o_ref, *, t_block):
    z = x_ref[0]
    n_insts = z.shape[0]
    tau_i = _bisect_tau(z, axis=0)
    pi = jnp.maximum(z - tau_i, 0.0)
    nf = t_block // _LST
    zt = z.T.reshape(nf, _LST, n_insts)
    tau_t = _bisect_tau(zt, axis=1)
    pt = jnp.maximum(zt - tau_t, 0.0).reshape(t_block, n_insts).T
    o_ref[0] = pi * pt


def kernel(midis_out):
    batch, n_insts, time = midis_out.shape
    t_block = 2048
    if time % t_block:
        t_block = _LST
    grid = (batch, time // t_block)
    spec = pl.BlockSpec((1, n_insts, t_block), lambda b, t: (b, 0, t))
    return pl.pallas_call(
        functools.partial(_body, t_block=t_block),
        grid=grid,
        in_specs=[spec],
        out_specs=spec,
        out_shape=jax.ShapeDtypeStruct(midis_out.shape, midis_out.dtype),
    )(midis_out)


# speedup vs baseline: 14.2131x; 1.3236x over previous
import functools

import jax
import jax.numpy as jnp
from jax.experimental import pallas as pl

_LST = 64
_NB = 10


def _bisect_tau(z, axis):
    hi = jnp.max(z, axis=axis, keepdims=True)
    lo = hi - 1.0
    for _ in range(_NB):
        mid = 0.5 * (lo + hi)
        g = jnp.sum(jnp.maximum(z - mid, 0.0), axis=axis, keepdims=True)
        ge = g >= 1.0
        lo = jnp.where(ge, mid, lo)
        hi = jnp.where(ge, hi, mid)
    t = lo
    for _ in range(2):
        sup = (z > t).astype(jnp.float32)
        c = jnp.sum(sup, axis=axis, keepdims=True)
        s = jnp.sum(z * sup, axis=axis, keepdims=True)
        t = jnp.clip((s - 1.0) / c, lo, hi)
    return t


def _body(x_ref, o_ref, *, t_block):
    z = x_ref[0]
    n_insts = z.shape[0]
    tau_i = _bisect_tau(z, axis=0)
    pi = jnp.maximum(z - tau_i, 0.0)
    nf = t_block // _LST
    zt = z.T.reshape(nf, _LST, n_insts)
    tau_t = _bisect_tau(zt, axis=1)
    pt = jnp.maximum(zt - tau_t, 0.0).reshape(t_block, n_insts).T
    o_ref[0] = pi * pt


def kernel(midis_out):
    batch, n_insts, time = midis_out.shape
    t_block = 2048
    if time % t_block:
        t_block = _LST
    grid = (batch, time // t_block)
    spec = pl.BlockSpec((1, n_insts, t_block), lambda b, t: (b, 0, t))
    return pl.pallas_call(
        functools.partial(_body, t_block=t_block),
        grid=grid,
        in_specs=[spec],
        out_specs=spec,
        out_shape=jax.ShapeDtypeStruct(midis_out.shape, midis_out.dtype),
    )(midis_out)


# speedup vs baseline: 20.7283x; 1.4584x over previous
import functools

import jax
import jax.numpy as jnp
from jax.experimental import pallas as pl

_LST = 64
_NB = 9


def _bisect_tau(z, axis):
    d = float(z.shape[axis])
    hi = jnp.max(z, axis=axis, keepdims=True)
    lo = hi - 1.0
    for _ in range(_NB):
        mid = 0.5 * (lo + hi)
        g = jnp.sum(jnp.maximum(z, mid), axis=axis, keepdims=True)
        ge = g >= 1.0 + d * mid
        lo = jnp.where(ge, mid, lo)
        hi = jnp.where(ge, hi, mid)
    sup = (z > lo).astype(jnp.float32)
    c = jnp.sum(sup, axis=axis, keepdims=True)
    s = jnp.sum(z * sup, axis=axis, keepdims=True)
    return jnp.clip((s - 1.0) / c, lo, hi)


def _body(x_ref, o_ref, *, t_block):
    z = x_ref[0]
    n_insts = z.shape[0]
    tau_i = _bisect_tau(z, axis=0)
    pi = jnp.maximum(z - tau_i, 0.0)
    nf = t_block // _LST
    zt = z.T.reshape(nf, _LST, n_insts)
    tau_t = _bisect_tau(zt, axis=1)
    pt = jnp.maximum(zt - tau_t, 0.0).reshape(t_block, n_insts).T
    o_ref[0] = pi * pt


def kernel(midis_out):
    batch, n_insts, time = midis_out.shape
    t_block = 2048
    if time % t_block:
        t_block = _LST
    grid = (batch, time // t_block)
    spec = pl.BlockSpec((1, n_insts, t_block), lambda b, t: (b, 0, t))
    return pl.pallas_call(
        functools.partial(_body, t_block=t_block),
        grid=grid,
        in_specs=[spec],
        out_specs=spec,
        out_shape=jax.ShapeDtypeStruct(midis_out.shape, midis_out.dtype),
    )(midis_out)
